# baseline (speedup 1.0000x reference)
import jax
import jax.numpy as jnp
from jax.experimental import pallas as pl

OBS = 8
PRED = 12
N = 512
H = 16
A_BLK = 128


def _lrelu(x):
    return jnp.where(x >= 0, x, 0.01 * x)


def _prep_kernel(traj_ref, mask_ref, pos_ref, WdecT_ref, bdec_ref,
                 WspT_ref, bsp_ref, Wm1aT_ref, Wm1bT_ref, bm1_ref,
                 pred_ref, u_ref, v_ref):
    M2 = jnp.dot(WspT_ref[...], Wm1aT_ref[...],
                 preferred_element_type=jnp.float32)
    c = jnp.dot(bsp_ref[...], Wm1aT_ref[...],
                preferred_element_type=jnp.float32) + bm1_ref[...]
    pred = traj_ref[...] * mask_ref[...]
    pred_ref[...] = pred
    P = pos_ref[...]
    for j in range(PRED):
        pj = pred[j]
        e = _lrelu(jnp.dot(pj, WdecT_ref[...],
                           preferred_element_type=jnp.float32) + bdec_ref[...])
        pm = jnp.dot(P, M2, preferred_element_type=jnp.float32)
        u_ref[j] = pm + c
        v_ref[j] = jnp.dot(e, Wm1bT_ref[...],
                           preferred_element_type=jnp.float32) - pm
        P = P + pj


def _pool_kernel(u_ref, v_ref, nei_ref, Wm2T_ref, bm2_ref, out_ref):
    u = u_ref[0]
    v = v_ref[0]
    z = jnp.maximum(u[:, None, :] + v[None, :, :], 0.0)
    h = jnp.dot(z.reshape(A_BLK * N, 64), Wm2T_ref[...],
                preferred_element_type=jnp.float32) + bm2_ref[...]
    h = jnp.maximum(h, 0.0).reshape(A_BLK, N, H)
    neg = jnp.where(nei_ref[0] > 0, 0.0, -1e30)
    hm = h + jax.lax.broadcast_in_dim(neg, (A_BLK, N, H), (0, 1))
    pool = jnp.max(hm, axis=1)
    out_ref[0] = jnp.where(pool <= -1e29, 0.0, pool)


def _head_kernel(rel_ref, inter_ref, Wl1aT_ref, Wl1bT_ref, bl1_ref,
                 Wl2T_ref, bl2_ref, WclsT_ref, bcls_ref, out_ref):
    x1 = _lrelu(jnp.dot(rel_ref[...], Wl1aT_ref[...],
                        preferred_element_type=jnp.float32)
                + jnp.dot(inter_ref[...], Wl1bT_ref[...],
                          preferred_element_type=jnp.float32)
                + bl1_ref[...])
    x2 = _lrelu(jnp.dot(x1, Wl2T_ref[...],
                        preferred_element_type=jnp.float32) + bl2_ref[...])
    out_ref[...] = jnp.dot(x2, WclsT_ref[...],
                           preferred_element_type=jnp.float32) + bcls_ref[...]


def kernel(obs_rel, traj_rel_pred, obs_traj_pos, nei_index, nei_num_index,
           loss_mask, W_dec, b_dec, W_sp, b_sp, W_m1, b_m1, W_m2, b_m2,
           W_l1, b_l1, W_l2, b_l2, W_cls, b_cls):
    f32 = jnp.float32
    pos = obs_traj_pos[-1]

    pred, U, V = pl.pallas_call(
        _prep_kernel,
        out_shape=[
            jax.ShapeDtypeStruct((PRED, N, 2), f32),
            jax.ShapeDtypeStruct((PRED, N, 64), f32),
            jax.ShapeDtypeStruct((PRED, N, 64), f32),
        ],
    )(traj_rel_pred, loss_mask, pos, W_dec.T, b_dec.reshape(1, H),
      W_sp.T, b_sp.reshape(1, H), W_m1[:, :H].T, W_m1[:, H:].T,
      b_m1.reshape(1, 64))

    pool = pl.pallas_call(
        _pool_kernel,
        grid=(PRED, N // A_BLK),
        in_specs=[
            pl.BlockSpec((1, A_BLK, 64), lambda j, a: (j, a, 0)),
            pl.BlockSpec((1, N, 64), lambda j, a: (j, 0, 0)),
            pl.BlockSpec((1, A_BLK, N), lambda j, a: (j, a, 0)),
            pl.BlockSpec((64, H), lambda j, a: (0, 0)),
            pl.BlockSpec((1, H), lambda j, a: (0, 0)),
        ],
        out_specs=pl.BlockSpec((1, A_BLK, H), lambda j, a: (j, a, 0)),
        out_shape=jax.ShapeDtypeStruct((PRED, N, H), f32),
    )(U, V, nei_index, W_m2.T, b_m2.reshape(1, H))

    rel = jnp.concatenate([obs_rel, pred], axis=0)
    rel = jnp.transpose(rel, (1, 0, 2)).reshape(N, 2 * (OBS + PRED))
    inter = jnp.transpose(pool, (1, 0, 2)).reshape(N, H * PRED)

    D_rel = 2 * (OBS + PRED)
    cls = pl.pallas_call(
        _head_kernel,
        out_shape=jax.ShapeDtypeStruct((N, 1), f32),
    )(rel, inter, W_l1[:, :D_rel].T, W_l1[:, D_rel:].T, b_l1.reshape(1, 32),
      W_l2.T, b_l2.reshape(1, 64), W_cls.T, b_cls.reshape(1, 1))
    return cls


# speedup vs baseline: 1.2404x; 1.2404x over previous
import jax
import jax.numpy as jnp
from jax.experimental import pallas as pl
from jax.experimental.pallas import tpu as pltpu

OBS = 8
PRED = 12
N = 512
H = 16
A_BLK = 128
G = N // 8


def _lrelu(x):
    return jnp.where(x >= 0, x, 0.01 * x)


def _prep_kernel(traj_ref, mask_ref, pos_ref, WdecT_ref, bdec_ref,
                 WspT_ref, bsp_ref, Wm1aT_ref, Wm1bT_ref, bm1_ref,
                 Wm2T_ref, bm2_ref,
                 pred_ref, u_ref, v_ref, wbig_ref, b2l_ref):
    M2 = jnp.dot(WspT_ref[...], Wm1aT_ref[...],
                 preferred_element_type=jnp.float32)
    c = jnp.dot(bsp_ref[...], Wm1aT_ref[...],
                preferred_element_type=jnp.float32) + bm1_ref[...]
    pred = traj_ref[...] * mask_ref[...]
    pred_ref[...] = pred
    P = pos_ref[...]
    for j in range(PRED):
        pj = pred[j]
        e = _lrelu(jnp.dot(pj, WdecT_ref[...],
                           preferred_element_type=jnp.float32) + bdec_ref[...])
        pm = jnp.dot(P, M2, preferred_element_type=jnp.float32)
        u_ref[j] = pm + c
        v_ref[j] = jnp.dot(e, Wm1bT_ref[...],
                           preferred_element_type=jnp.float32) - pm
        P = P + pj
    lane16 = jax.lax.broadcasted_iota(jnp.int32, (H, 8 * H), 1)
    d16 = jax.lax.broadcasted_iota(jnp.int32, (H, 8 * H), 0)
    T2 = (lane16 % H == d16).astype(jnp.float32)
    wrep = jnp.dot(Wm2T_ref[...], T2,
                   preferred_element_type=jnp.float32)
    wtile = jnp.concatenate([wrep] * 8, axis=0)
    r_iota = jax.lax.broadcasted_iota(jnp.int32, (N, 8 * H), 0)
    l_iota = jax.lax.broadcasted_iota(jnp.int32, (N, 8 * H), 1)
    wbig_ref[...] = jnp.where(r_iota // 64 == l_iota // H, wtile, 0.0)
    b2l_ref[...] = jnp.dot(bm2_ref[...], T2,
                           preferred_element_type=jnp.float32)


def _pool_kernel(u_ref, v8_ref, nei4_ref, wbig_ref, b2l_ref, out_ref):
    u = u_ref[0]
    u_t = jnp.concatenate([u] * 8, axis=1)
    v8 = v8_ref[0]
    z = jnp.maximum(u_t[:, None, :] + v8[None, :, :], 0.0)
    h = jnp.dot(z.reshape(A_BLK * G, 8 * 64), wbig_ref[...],
                preferred_element_type=jnp.float32) + b2l_ref[...]
    h = jnp.maximum(h, 0.0)
    neif = (nei4_ref[0] > 0).astype(jnp.float32).reshape(A_BLK * G, 8)
    j_iota = jax.lax.broadcasted_iota(jnp.int32, (8, 8 * H), 0)
    l2 = jax.lax.broadcasted_iota(jnp.int32, (8, 8 * H), 1)
    S = (l2 // H == j_iota).astype(jnp.float32)
    mL = jnp.dot(neif, S, preferred_element_type=jnp.float32)
    hm = (h * mL).reshape(A_BLK, G, 8 * H)
    q = jnp.max(hm, axis=1)
    q = jnp.maximum(q, pltpu.roll(q, 64, 1))
    q = jnp.maximum(q, pltpu.roll(q, 32, 1))
    q = jnp.maximum(q, pltpu.roll(q, 16, 1))
    out_ref[0] = q[:, :H]


def _head_kernel(rel_ref, inter_ref, Wl1aT_ref, Wl1bT_ref, bl1_ref,
                 Wl2T_ref, bl2_ref, WclsT_ref, bcls_ref, out_ref):
    x1 = _lrelu(jnp.dot(rel_ref[...], Wl1aT_ref[...],
                        preferred_element_type=jnp.float32)
                + jnp.dot(inter_ref[...], Wl1bT_ref[...],
                          preferred_element_type=jnp.float32)
                + bl1_ref[...])
    x2 = _lrelu(jnp.dot(x1, Wl2T_ref[...],
                        preferred_element_type=jnp.float32) + bl2_ref[...])
    out_ref[...] = jnp.dot(x2, WclsT_ref[...],
                           preferred_element_type=jnp.float32) + bcls_ref[...]


def kernel(obs_rel, traj_rel_pred, obs_traj_pos, nei_index, nei_num_index,
           loss_mask, W_dec, b_dec, W_sp, b_sp, W_m1, b_m1, W_m2, b_m2,
           W_l1, b_l1, W_l2, b_l2, W_cls, b_cls):
    f32 = jnp.float32
    pos = obs_traj_pos[-1]

    pred, U, V, Wbig, b2L = pl.pallas_call(
        _prep_kernel,
        out_shape=[
            jax.ShapeDtypeStruct((PRED, N, 2), f32),
            jax.ShapeDtypeStruct((PRED, N, 64), f32),
            jax.ShapeDtypeStruct((PRED, N, 64), f32),
            jax.ShapeDtypeStruct((N, 8 * H), f32),
            jax.ShapeDtypeStruct((1, 8 * H), f32),
        ],
    )(traj_rel_pred, loss_mask, pos, W_dec.T, b_dec.reshape(1, H),
      W_sp.T, b_sp.reshape(1, H), W_m1[:, :H].T, W_m1[:, H:].T,
      b_m1.reshape(1, 64), W_m2.T, b_m2.reshape(1, H))

    V8 = V.reshape(PRED, G, 8 * 64)
    nei4 = nei_index.reshape(PRED, N, G, 8)

    pool = pl.pallas_call(
        _pool_kernel,
        grid=(PRED, N // A_BLK),
        in_specs=[
            pl.BlockSpec((1, A_BLK, 64), lambda j, a: (j, a, 0)),
            pl.BlockSpec((1, G, 8 * 64), lambda j, a: (j, 0, 0)),
            pl.BlockSpec((1, A_BLK, G, 8), lambda j, a: (j, a, 0, 0)),
            pl.BlockSpec((N, 8 * H), lambda j, a: (0, 0)),
            pl.BlockSpec((1, 8 * H), lambda j, a: (0, 0)),
        ],
        out_specs=pl.BlockSpec((1, A_BLK, H), lambda j, a: (j, a, 0)),
        out_shape=jax.ShapeDtypeStruct((PRED, N, H), f32),
    )(U, V8, nei4, Wbig, b2L)

    rel = jnp.concatenate([obs_rel, pred], axis=0)
    rel = jnp.transpose(rel, (1, 0, 2)).reshape(N, 2 * (OBS + PRED))
    inter = jnp.transpose(pool, (1, 0, 2)).reshape(N, H * PRED)

    D_rel = 2 * (OBS + PRED)
    cls = pl.pallas_call(
        _head_kernel,
        out_shape=jax.ShapeDtypeStruct((N, 1), f32),
    )(rel, inter, W_l1[:, :D_rel].T, W_l1[:, D_rel:].T, b_l1.reshape(1, 32),
      W_l2.T, b_l2.reshape(1, 64), W_cls.T, b_cls.reshape(1, 1))
    return cls


# speedup vs baseline: 1.9548x; 1.5760x over previous
import jax
import jax.numpy as jnp
from jax.experimental import pallas as pl
from jax.experimental.pallas import tpu as pltpu

OBS = 8
PRED = 12
N = 512
H = 16
A_BLK = 128
G = N // 8


def _lrelu(x):
    return jnp.where(x >= 0, x, 0.01 * x)


def _prep_kernel(traj_ref, mask_ref, pos_ref, WdecT_ref, bdec_ref,
                 WspT_ref, bsp_ref, Wm1aT_ref, Wm1bT_ref, bm1_ref,
                 Wm2T_ref, bm2_ref,
                 pred_ref, u_ref, v_ref, wbig_ref, b2l_ref):
    M2 = jnp.dot(WspT_ref[...], Wm1aT_ref[...],
                 preferred_element_type=jnp.float32)
    c = jnp.dot(bsp_ref[...], Wm1aT_ref[...],
                preferred_element_type=jnp.float32) + bm1_ref[...]
    pred = traj_ref[...] * mask_ref[...]
    pred_ref[...] = pred
    P = pos_ref[...]
    for j in range(PRED):
        pj = pred[j]
        e = _lrelu(jnp.dot(pj, WdecT_ref[...],
                           preferred_element_type=jnp.float32) + bdec_ref[...])
        pm = jnp.dot(P, M2, preferred_element_type=jnp.float32)
        u_ref[j] = pm + c
        v_ref[j] = jnp.dot(e, Wm1bT_ref[...],
                           preferred_element_type=jnp.float32) - pm
        P = P + pj
    lane16 = jax.lax.broadcasted_iota(jnp.int32, (H, 8 * H), 1)
    d16 = jax.lax.broadcasted_iota(jnp.int32, (H, 8 * H), 0)
    T2 = (lane16 % H == d16).astype(jnp.float32)
    wrep = jnp.dot(Wm2T_ref[...], T2,
                   preferred_element_type=jnp.float32)
    wtile = jnp.concatenate([wrep] * 8, axis=0)
    r_iota = jax.lax.broadcasted_iota(jnp.int32, (N, 8 * H), 0)
    l_iota = jax.lax.broadcasted_iota(jnp.int32, (N, 8 * H), 1)
    wbig_ref[...] = jnp.where(r_iota // 64 == l_iota // H, wtile, 0.0)
    b2l_ref[...] = jnp.dot(bm2_ref[...], T2,
                           preferred_element_type=jnp.float32)


def _pool_kernel(u_ref, v8_ref, nei_ref, wbig_ref, b2l_ref, out_ref):
    u = u_ref[0]
    u_t = jnp.concatenate([u] * 8, axis=1)
    v8 = v8_ref[0]
    z = jnp.maximum(u_t[:, None, :] + v8[None, :, :], 0.0)
    h = jnp.dot(z.reshape(A_BLK * G, 8 * 64), wbig_ref[...],
                preferred_element_type=jnp.float32) + b2l_ref[...]
    h = jnp.maximum(h, 0.0)
    m3 = nei_ref[0].reshape(A_BLK, G, 8)
    neif = (m3 > 0).astype(jnp.float32)
    m8 = neif.reshape(A_BLK * G, 8)
    j_iota = jax.lax.broadcasted_iota(jnp.int32, (8, 8 * H), 0)
    l2 = jax.lax.broadcasted_iota(jnp.int32, (8, 8 * H), 1)
    S = (l2 // H == j_iota).astype(jnp.float32)
    mL = jnp.dot(m8, S, preferred_element_type=jnp.float32)
    hm = (h * mL).reshape(A_BLK, G, 8 * H)
    q = jnp.max(hm, axis=1)
    q = jnp.maximum(q, pltpu.roll(q, 64, 1))
    q = jnp.maximum(q, pltpu.roll(q, 32, 1))
    q = jnp.maximum(q, pltpu.roll(q, 16, 1))
    out_ref[0] = q[:, :H]


def _head_kernel(rel_ref, inter_ref, Wl1aT_ref, Wl1bT_ref, bl1_ref,
                 Wl2T_ref, bl2_ref, WclsT_ref, bcls_ref, out_ref):
    x1 = _lrelu(jnp.dot(rel_ref[...], Wl1aT_ref[...],
                        preferred_element_type=jnp.float32)
                + jnp.dot(inter_ref[...], Wl1bT_ref[...],
                          preferred_element_type=jnp.float32)
                + bl1_ref[...])
    x2 = _lrelu(jnp.dot(x1, Wl2T_ref[...],
                        preferred_element_type=jnp.float32) + bl2_ref[...])
    out_ref[...] = jnp.dot(x2, WclsT_ref[...],
                           preferred_element_type=jnp.float32) + bcls_ref[...]


def kernel(obs_rel, traj_rel_pred, obs_traj_pos, nei_index, nei_num_index,
           loss_mask, W_dec, b_dec, W_sp, b_sp, W_m1, b_m1, W_m2, b_m2,
           W_l1, b_l1, W_l2, b_l2, W_cls, b_cls):
    f32 = jnp.float32
    pos = obs_traj_pos[-1]

    pred, U, V, Wbig, b2L = pl.pallas_call(
        _prep_kernel,
        out_shape=[
            jax.ShapeDtypeStruct((PRED, N, 2), f32),
            jax.ShapeDtypeStruct((PRED, N, 64), f32),
            jax.ShapeDtypeStruct((PRED, N, 64), f32),
            jax.ShapeDtypeStruct((N, 8 * H), f32),
            jax.ShapeDtypeStruct((1, 8 * H), f32),
        ],
    )(traj_rel_pred, loss_mask, pos, W_dec.T, b_dec.reshape(1, H),
      W_sp.T, b_sp.reshape(1, H), W_m1[:, :H].T, W_m1[:, H:].T,
      b_m1.reshape(1, 64), W_m2.T, b_m2.reshape(1, H))

    V8 = V.reshape(PRED, G, 8 * 64)

    pool = pl.pallas_call(
        _pool_kernel,
        grid=(PRED, N // A_BLK),
        in_specs=[
            pl.BlockSpec((1, A_BLK, 64), lambda j, a: (j, a, 0)),
            pl.BlockSpec((1, G, 8 * 64), lambda j, a: (j, 0, 0)),
            pl.BlockSpec((1, A_BLK, N), lambda j, a: (j, a, 0)),
            pl.BlockSpec((N, 8 * H), lambda j, a: (0, 0)),
            pl.BlockSpec((1, 8 * H), lambda j, a: (0, 0)),
        ],
        out_specs=pl.BlockSpec((1, A_BLK, H), lambda j, a: (j, a, 0)),
        out_shape=jax.ShapeDtypeStruct((PRED, N, H), f32),
    )(U, V8, nei_index, Wbig, b2L)

    rel = jnp.concatenate([obs_rel, pred], axis=0)
    rel = jnp.transpose(rel, (1, 0, 2)).reshape(N, 2 * (OBS + PRED))
    inter = jnp.transpose(pool, (1, 0, 2)).reshape(N, H * PRED)

    D_rel = 2 * (OBS + PRED)
    cls = pl.pallas_call(
        _head_kernel,
        out_shape=jax.ShapeDtypeStruct((N, 1), f32),
    )(rel, inter, W_l1[:, :D_rel].T, W_l1[:, D_rel:].T, b_l1.reshape(1, 32),
      W_l2.T, b_l2.reshape(1, 64), W_cls.T, b_cls.reshape(1, 1))
    return cls
